# baseline (speedup 1.0000x reference)
import functools

import jax
import jax.numpy as jnp
from jax import lax
from jax.experimental import pallas as pl
from jax.experimental.pallas import tpu as pltpu
from jax.experimental.pallas import tpu_sc as plsc

N_EMBED = 8192
DIM = 32
N_TOK = 16384


TOK_BLOCK = 256


def _he2_body(et_ref, he2_ref):
    et = et_ref[...]
    he2_ref[...] = 0.5 * jnp.sum(et * et, axis=0, keepdims=True)


def _argmin_body(x_ref, et_ref, he2_ref, idx_ref):
    xt = x_ref[...]
    prod = lax.dot_general(xt, et_ref[...], (((1,), (0,)), ((), ())),
                           preferred_element_type=jnp.float32)
    h = he2_ref[...] - prod
    idx_ref[...] = jnp.argmin(h, axis=1).astype(jnp.int32)


def _argmin_indices(flat_x, embed_t, he2):
    n = flat_x.shape[0]
    grid = n // TOK_BLOCK
    return pl.pallas_call(
        _argmin_body,
        grid=(grid,),
        in_specs=[
            pl.BlockSpec((TOK_BLOCK, DIM), lambda i: (i, 0)),
            pl.BlockSpec((DIM, N_EMBED), lambda i: (0, 0)),
            pl.BlockSpec((1, N_EMBED), lambda i: (0, 0)),
        ],
        out_specs=pl.BlockSpec((TOK_BLOCK,), lambda i: (i,)),
        out_shape=jax.ShapeDtypeStruct((n,), jnp.int32),
    )(flat_x, embed_t, he2)


NC, NS, LANES = 2, 16, 16
NW = NC * NS
CHUNK = 128


def _make_gather_body(n_chunk, b_per_w):
    def _gather_body(table_hbm, idx_hbm, out_hbm, idx_v, rows_v, sem):
        wid = lax.axis_index("s") * NC + lax.axis_index("c")
        pltpu.sync_copy(idx_hbm.at[pl.ds(wid * n_chunk, n_chunk)], idx_v)
        copies = [
            pltpu.async_copy(table_hbm.at[idx_v.at[j]],
                             rows_v.at[pl.ds(j * CHUNK, CHUNK)], sem)
            for j in range(n_chunk)
        ]
        for c in copies:
            c.wait()
        pltpu.sync_copy(rows_v, out_hbm.at[pl.ds(wid * b_per_w, b_per_w)])
    return _gather_body


def _sc_gather(embed, idx):
    n = idx.shape[0]
    b_per_w = n // NW
    n_chunk = b_per_w // CHUNK
    mesh = plsc.VectorSubcoreMesh(core_axis_name="c", subcore_axis_name="s")
    f = functools.partial(
        pl.kernel,
        mesh=mesh,
        out_type=jax.ShapeDtypeStruct((n, DIM), jnp.float32),
        scratch_types=[
            pltpu.VMEM((n_chunk, CHUNK), jnp.int32),
            pltpu.VMEM((b_per_w, DIM), jnp.float32),
            pltpu.SemaphoreType.DMA,
        ],
        compiler_params=pltpu.CompilerParams(use_tc_tiling_on_sc=False),
    )(_make_gather_body(n_chunk, b_per_w))
    return f(embed, idx.reshape(NW * n_chunk, CHUNK))


def kernel(x, embed):
    flat_x = x.reshape(-1, DIM)
    embed_t = embed.T
    he2 = pl.pallas_call(
        _he2_body,
        out_shape=jax.ShapeDtypeStruct((1, N_EMBED), jnp.float32),
    )(embed_t)
    idx = _argmin_indices(flat_x, embed_t, he2)
    return _sc_gather(embed, idx).reshape(x.shape)


# speedup vs baseline: 1.0321x; 1.0321x over previous
import functools

import jax
import jax.numpy as jnp
from jax import lax
from jax.experimental import pallas as pl
from jax.experimental.pallas import tpu as pltpu
from jax.experimental.pallas import tpu_sc as plsc

N_EMBED = 8192
DIM = 32
N_TOK = 16384


TOK_BLOCK = 512


def _he2_body(et_ref, he2_ref):
    et = et_ref[...]
    he2_ref[...] = 0.5 * jnp.sum(et * et, axis=0, keepdims=True)


def _argmin_body(x_ref, et_ref, he2_ref, idx_ref):
    xt = x_ref[...]
    prod = lax.dot_general(xt, et_ref[...], (((1,), (0,)), ((), ())),
                           preferred_element_type=jnp.float32)
    h = he2_ref[...] - prod
    idx_ref[...] = jnp.argmin(h, axis=1).astype(jnp.int32)


def _argmin_indices(flat_x, embed_t, he2):
    n = flat_x.shape[0]
    grid = n // TOK_BLOCK
    return pl.pallas_call(
        _argmin_body,
        grid=(grid,),
        in_specs=[
            pl.BlockSpec((TOK_BLOCK, DIM), lambda i: (i, 0)),
            pl.BlockSpec((DIM, N_EMBED), lambda i: (0, 0)),
            pl.BlockSpec((1, N_EMBED), lambda i: (0, 0)),
        ],
        out_specs=pl.BlockSpec((TOK_BLOCK,), lambda i: (i,)),
        out_shape=jax.ShapeDtypeStruct((n,), jnp.int32),
    )(flat_x, embed_t, he2)


NC, NS, LANES = 2, 16, 16
NW = NC * NS
CHUNK = 128


def _make_gather_body(n_chunk, b_per_w):
    def _gather_body(table_hbm, idx_hbm, out_hbm, idx_v, rows_v, sem):
        wid = lax.axis_index("s") * NC + lax.axis_index("c")
        pltpu.sync_copy(idx_hbm.at[pl.ds(wid * n_chunk, n_chunk)], idx_v)
        copies = [
            pltpu.async_copy(table_hbm.at[idx_v.at[j]],
                             rows_v.at[pl.ds(j * CHUNK, CHUNK)], sem)
            for j in range(n_chunk)
        ]
        for c in copies:
            c.wait()
        pltpu.sync_copy(rows_v, out_hbm.at[pl.ds(wid * b_per_w, b_per_w)])
    return _gather_body


def _sc_gather(embed, idx):
    n = idx.shape[0]
    b_per_w = n // NW
    n_chunk = b_per_w // CHUNK
    mesh = plsc.VectorSubcoreMesh(core_axis_name="c", subcore_axis_name="s")
    f = functools.partial(
        pl.kernel,
        mesh=mesh,
        out_type=jax.ShapeDtypeStruct((n, DIM), jnp.float32),
        scratch_types=[
            pltpu.VMEM((n_chunk, CHUNK), jnp.int32),
            pltpu.VMEM((b_per_w, DIM), jnp.float32),
            pltpu.SemaphoreType.DMA,
        ],
        compiler_params=pltpu.CompilerParams(use_tc_tiling_on_sc=False),
    )(_make_gather_body(n_chunk, b_per_w))
    return f(embed, idx.reshape(NW * n_chunk, CHUNK))


def kernel(x, embed):
    flat_x = x.reshape(-1, DIM)
    embed_t = embed.T
    he2 = pl.pallas_call(
        _he2_body,
        out_shape=jax.ShapeDtypeStruct((1, N_EMBED), jnp.float32),
    )(embed_t)
    idx = _argmin_indices(flat_x, embed_t, he2)
    return _sc_gather(embed, idx).reshape(x.shape)
